# baseline (speedup 1.0000x reference)
import jax
import jax.numpy as jnp
from jax.experimental import pallas as pl
from jax.experimental.pallas import tpu as pltpu

_BLOCK_SIZE = 8192
_N_EMBD = 1024
_CHUNK = 512
_NBUF = 4
_NCHUNKS = _BLOCK_SIZE // _CHUNK


def _ring_body(x_ref, o_ref, buf, *sems):
    sin = sems[:_NBUF]
    sout = sems[_NBUF:]

    def cin(i):
        b = i % _NBUF
        return pltpu.make_async_copy(
            x_ref.at[pl.ds(i * _CHUNK, _CHUNK)], buf.at[b], sin[b]
        )

    def cout(i):
        b = i % _NBUF
        return pltpu.make_async_copy(
            buf.at[b], o_ref.at[pl.ds(i * _CHUNK, _CHUNK)], sout[b]
        )

    ins = [None] * _NCHUNKS
    outs = [None] * _NCHUNKS
    for i in range(_NCHUNKS):
        if i >= _NBUF:
            outs[i - _NBUF].wait()
        ins[i] = cin(i)
        ins[i].start()
        if i >= 1:
            ins[i - 1].wait()
            outs[i - 1] = cout(i - 1)
            outs[i - 1].start()
    ins[_NCHUNKS - 1].wait()
    outs[_NCHUNKS - 1] = cout(_NCHUNKS - 1)
    outs[_NCHUNKS - 1].start()
    for j in range(_NCHUNKS - _NBUF, _NCHUNKS):
        outs[j].wait()


def kernel(wpe):
    out = pl.pallas_call(
        _ring_body,
        in_specs=[pl.BlockSpec(memory_space=pl.ANY)],
        out_specs=pl.BlockSpec(memory_space=pl.ANY),
        out_shape=jax.ShapeDtypeStruct((_BLOCK_SIZE, _N_EMBD), jnp.float32),
        scratch_shapes=(
            [pltpu.VMEM((_NBUF, _CHUNK, _N_EMBD), jnp.float32)]
            + [pltpu.SemaphoreType.DMA] * (2 * _NBUF)
        ),
    )(wpe)
    return out[None]


# speedup vs baseline: 1.1227x; 1.1227x over previous
import jax
import jax.numpy as jnp
from jax.experimental import pallas as pl
from jax.experimental.pallas import tpu as pltpu

_BLOCK_SIZE = 8192
_N_EMBD = 1024
_CHUNK = 2048
_NBUF = 4
_NCHUNKS = _BLOCK_SIZE // _CHUNK


def _ring_body(x_ref, o_ref, buf, *sems):
    sin = sems[:_NBUF]
    sout = sems[_NBUF:]

    def cin(i):
        b = i % _NBUF
        return pltpu.make_async_copy(
            x_ref.at[pl.ds(i * _CHUNK, _CHUNK)], buf.at[b], sin[b]
        )

    def cout(i):
        b = i % _NBUF
        return pltpu.make_async_copy(
            buf.at[b], o_ref.at[pl.ds(i * _CHUNK, _CHUNK)], sout[b]
        )

    ins = [None] * _NCHUNKS
    outs = [None] * _NCHUNKS
    for i in range(_NCHUNKS):
        if i >= _NBUF:
            outs[i - _NBUF].wait()
        ins[i] = cin(i)
        ins[i].start()
        if i >= 1:
            ins[i - 1].wait()
            outs[i - 1] = cout(i - 1)
            outs[i - 1].start()
    ins[_NCHUNKS - 1].wait()
    outs[_NCHUNKS - 1] = cout(_NCHUNKS - 1)
    outs[_NCHUNKS - 1].start()
    for j in range(_NCHUNKS - _NBUF, _NCHUNKS):
        outs[j].wait()


def kernel(wpe):
    out = pl.pallas_call(
        _ring_body,
        in_specs=[pl.BlockSpec(memory_space=pl.ANY)],
        out_specs=pl.BlockSpec(memory_space=pl.ANY),
        out_shape=jax.ShapeDtypeStruct((_BLOCK_SIZE, _N_EMBD), jnp.float32),
        scratch_shapes=(
            [pltpu.VMEM((_NBUF, _CHUNK, _N_EMBD), jnp.float32)]
            + [pltpu.SemaphoreType.DMA] * (2 * _NBUF)
        ),
    )(wpe)
    return out[None]
